# baseline (speedup 1.0000x reference)
import functools

import jax
import jax.numpy as jnp
from jax import lax
from jax.experimental import pallas as pl
from jax.experimental.pallas import tpu as pltpu
from jax.experimental.pallas import tpu_sc as plsc

N_BINS_K = 4096
BATCH_K = 16384
L = 16
NW = 32
SC_BATCH = 8192
TC_BATCH = BATCH_K - SC_BATCH
ROWS_PER_W = SC_BATCH // NW
N_TIMES = N_BINS_K + 1
N_TIMES_P2 = N_TIMES + 127


def _pwc_sc_body(times_hbm, t_hbm, values_hbm, out_hbm,
                 times_v, t_v, tile_v, col_v, sem_a, sem_b, sem_c, sem_d):
    wid = lax.axis_index("s") * 2 + lax.axis_index("c")
    base = wid * ROWS_PER_W
    base8 = pl.multiple_of(base, 8)

    cp_times = pltpu.async_copy(times_hbm, times_v, sem_a)
    cp_t = pltpu.async_copy(t_hbm, t_v, sem_b)
    cp_spec = pltpu.async_copy(
        values_hbm.at[pl.ds(base8, ROWS_PER_W), pl.ds(0, 128)],
        tile_v, sem_c)
    cp_times.wait()
    cp_t.wait()

    lanes = lax.iota(jnp.int32, L)
    zeros = jnp.zeros((L,), jnp.int32)
    tval = plsc.load_gather(t_v, [zeros])

    def probe(idx):
        smp = plsc.load_gather(times_v, [jnp.minimum(idx, N_BINS_K)])
        ok = jnp.logical_and(smp <= tval, idx <= N_BINS_K)
        return jnp.sum(jnp.where(ok, jnp.int32(1), jnp.int32(0)))

    n1 = probe(lanes * 256 + 255)
    b1 = n1 * 256
    n2 = probe(b1 + lanes * 16 + 15)
    b2 = b1 + n2 * 16
    n3 = probe(b2 + lanes)
    count = b2 + n3

    valid = jnp.logical_and(count >= 1, count <= N_BINS_K)
    col = jnp.clip(count - 1, 0, N_BINS_K - 1)

    col_tile = pl.multiple_of((col // 128) * 128, 128)
    cp_spec.wait()

    @pl.when(col >= 128)
    def _refetch():
        pltpu.async_copy(
            values_hbm.at[pl.ds(base8, ROWS_PER_W), pl.ds(col_tile, 128)],
            tile_v, sem_d).wait()

    vf = jnp.where(valid, jnp.float32(1.0), jnp.float32(0.0))
    colrem = jnp.full((L,), col % 128, jnp.int32)

    def sel_body(c, carry):
        rows = c * L + lanes
        col_v[pl.ds(c * L, L)] = (
            plsc.load_gather(tile_v, [rows, colrem]) * vf)
        return carry

    lax.fori_loop(0, ROWS_PER_W // L, sel_body, jnp.int32(0))
    pltpu.sync_copy(col_v, out_hbm.at[pl.ds(base, ROWS_PER_W)])


def _pwc_tc_body(times_ref, t_ref, values_ref, out_ref, scr_ref, sem):
    tt = t_ref[0]
    cnt = jnp.sum(jnp.where(times_ref[...] <= tt, jnp.int32(1), jnp.int32(0)))
    valid = jnp.logical_and(cnt >= 1, cnt <= N_BINS_K)
    col = jnp.clip(cnt - 1, 0, N_BINS_K - 1)
    col_tile = pl.multiple_of((col // 128) * 128, 128)
    cp = pltpu.make_async_copy(
        values_ref.at[pl.ds(SC_BATCH, TC_BATCH), pl.ds(col_tile, 128)],
        scr_ref, sem)
    cp.start()
    cp.wait()
    onehot = lax.broadcasted_iota(jnp.int32, (TC_BATCH, 128), 1) == (col % 128)
    vf = jnp.where(valid, jnp.float32(1.0), jnp.float32(0.0))
    out_ref[...] = jnp.sum(jnp.where(onehot, scr_ref[...], 0.0), axis=1) * vf


@jax.jit
def _pwc_hybrid(times, t1, values):
    mesh = plsc.VectorSubcoreMesh(core_axis_name="c", subcore_axis_name="s")
    sc_fn = functools.partial(
        pl.kernel,
        mesh=mesh,
        out_type=jax.ShapeDtypeStruct((SC_BATCH,), jnp.float32),
        scratch_types=[
            pltpu.VMEM((N_TIMES,), jnp.float32),
            pltpu.VMEM((1,), jnp.float32),
            pltpu.VMEM((ROWS_PER_W, 128), jnp.float32),
            pltpu.VMEM((ROWS_PER_W,), jnp.float32),
            pltpu.SemaphoreType.DMA,
            pltpu.SemaphoreType.DMA,
            pltpu.SemaphoreType.DMA,
            pltpu.SemaphoreType.DMA,
        ],
        compiler_params=pltpu.CompilerParams(needs_layout_passes=False),
    )(_pwc_sc_body)
    sc_out = sc_fn(times, t1, values)

    times_p2 = jnp.concatenate(
        [times, jnp.full((N_TIMES_P2 - N_TIMES,), jnp.inf, jnp.float32)]
    ).reshape(N_TIMES_P2 // 128, 128)
    tc_out = pl.pallas_call(
        _pwc_tc_body,
        out_shape=jax.ShapeDtypeStruct((TC_BATCH,), jnp.float32),
        in_specs=[
            pl.BlockSpec(memory_space=pltpu.MemorySpace.VMEM),
            pl.BlockSpec(memory_space=pltpu.MemorySpace.SMEM),
            pl.BlockSpec(memory_space=pltpu.MemorySpace.HBM),
        ],
        out_specs=pl.BlockSpec(memory_space=pltpu.MemorySpace.VMEM),
        scratch_shapes=[
            pltpu.VMEM((TC_BATCH, 128), jnp.float32),
            pltpu.SemaphoreType.DMA,
        ],
    )(times_p2, t1, values)

    return jnp.concatenate([sc_out, tc_out])


def kernel(times, values, t):
    return _pwc_hybrid(times, jnp.reshape(t, (1,)), values)


# speedup vs baseline: 1.1263x; 1.1263x over previous
import functools

import jax
import jax.numpy as jnp
from jax import lax
from jax.experimental import pallas as pl
from jax.experimental.pallas import tpu as pltpu
from jax.experimental.pallas import tpu_sc as plsc

N_BINS_K = 4096
BATCH_K = 16384
L = 16
NW = 32
ROWS_PER_W = BATCH_K // NW
N_TIMES = N_BINS_K + 1


def _pwc_body(times_hbm, t_hbm, values_hbm, out_hbm,
              times_v, t_v, tile_v, col_v, sem_a, sem_b, sem_c, sem_d):
    wid = lax.axis_index("s") * 2 + lax.axis_index("c")
    base = wid * ROWS_PER_W
    base8 = pl.multiple_of(base, 8)

    cp_times = pltpu.async_copy(times_hbm, times_v, sem_a)
    cp_t = pltpu.async_copy(t_hbm, t_v, sem_b)
    half = ROWS_PER_W // 2
    cp_spec0 = pltpu.async_copy(
        values_hbm.at[pl.ds(base8, half), pl.ds(0, 128)],
        tile_v.at[pl.ds(0, half), :], sem_c)
    cp_spec1 = pltpu.async_copy(
        values_hbm.at[pl.ds(pl.multiple_of(base + half, 8), half),
                      pl.ds(0, 128)],
        tile_v.at[pl.ds(half, half), :], sem_d)
    cp_times.wait()
    cp_t.wait()

    lanes = lax.iota(jnp.int32, L)
    zeros = jnp.zeros((L,), jnp.int32)
    tval = plsc.load_gather(t_v, [zeros])

    def probe(idx):
        smp = plsc.load_gather(times_v, [jnp.minimum(idx, N_BINS_K)])
        ok = jnp.logical_and(smp <= tval, idx <= N_BINS_K)
        return jnp.sum(jnp.where(ok, jnp.int32(1), jnp.int32(0)))

    n1 = probe(lanes * 256 + 255)
    b1 = n1 * 256
    n2 = probe(b1 + lanes * 16 + 15)
    b2 = b1 + n2 * 16
    n3 = probe(b2 + lanes)
    count = b2 + n3

    valid = jnp.logical_and(count >= 1, count <= N_BINS_K)
    col = jnp.clip(count - 1, 0, N_BINS_K - 1)

    col_tile = pl.multiple_of((col // 128) * 128, 128)
    cp_spec0.wait()
    cp_spec1.wait()

    @pl.when(col >= 128)
    def _refetch():
        pltpu.async_copy(
            values_hbm.at[pl.ds(base8, ROWS_PER_W), pl.ds(col_tile, 128)],
            tile_v, sem_c).wait()

    vf = jnp.where(valid, jnp.float32(1.0), jnp.float32(0.0))
    colrem = jnp.full((L,), col % 128, jnp.int32)

    def sel_body(c, carry):
        rows = c * L + lanes
        col_v[pl.ds(c * L, L)] = (
            plsc.load_gather(tile_v, [rows, colrem]) * vf)
        return carry

    lax.fori_loop(0, ROWS_PER_W // L, sel_body, jnp.int32(0))
    pltpu.sync_copy(col_v, out_hbm.at[pl.ds(base, ROWS_PER_W)])


@jax.jit
def _pwc_sc(times, t1, values):
    mesh = plsc.VectorSubcoreMesh(core_axis_name="c", subcore_axis_name="s")
    f = functools.partial(
        pl.kernel,
        mesh=mesh,
        out_type=jax.ShapeDtypeStruct((BATCH_K,), jnp.float32),
        scratch_types=[
            pltpu.VMEM((N_TIMES,), jnp.float32),
            pltpu.VMEM((1,), jnp.float32),
            pltpu.VMEM((ROWS_PER_W, 128), jnp.float32),
            pltpu.VMEM((ROWS_PER_W,), jnp.float32),
            pltpu.SemaphoreType.DMA,
            pltpu.SemaphoreType.DMA,
            pltpu.SemaphoreType.DMA,
            pltpu.SemaphoreType.DMA,
        ],
        compiler_params=pltpu.CompilerParams(needs_layout_passes=False),
    )(_pwc_body)
    return f(times, t1, values)


def kernel(times, values, t):
    return _pwc_sc(times, jnp.reshape(t, (1,)), values)
